# Initial kernel scaffold; baseline (speedup 1.0000x reference)
import jax
import jax.numpy as jnp
from jax.experimental import pallas as pl


def kernel(x, pos, edge_index, params):
    raise NotImplementedError("write your pallas kernel here")


# speedup vs baseline: 4.1681x; 4.1681x over previous
import functools

import jax
import jax.numpy as jnp
from jax.experimental import pallas as pl

N = 10000
NPAD = 10240
ROWT = 256
EDGET = 2048
K = 5


def _mm(a, b):
    return jnp.dot(a, b, preferred_element_type=jnp.float32)


def _bn(h):
    m = jnp.mean(h, axis=0, keepdims=True)
    v = jnp.mean((h - m) ** 2, axis=0, keepdims=True)
    return (h - m) / jnp.sqrt(v + 1e-5)


def _knn_body(rows_ref, pat_ref, n2c_ref, o_ref):
    i = pl.program_id(0)
    pr = rows_ref[...]
    pat = pat_ref[...]
    n2r = jnp.sum(pr * pr, axis=1, keepdims=True)
    d = n2r + n2c_ref[...] - 2.0 * _mm(pr, pat)
    col = jax.lax.broadcasted_iota(jnp.int32, (ROWT, NPAD), 1).astype(
        jnp.float32)
    rowg = (i * ROWT + jax.lax.broadcasted_iota(jnp.int32, (ROWT, NPAD), 0)
            ).astype(jnp.float32)
    d = jnp.where(col == rowg, 0.0, d)
    d = jnp.maximum(d, 0.0)
    d = jnp.where(col >= float(N), 1e30, d)
    cols = []
    for _ in range(K + 1):
        mn = jnp.min(d, axis=1, keepdims=True)
        idxf = jnp.min(jnp.where(d == mn, col, 1e9), axis=1, keepdims=True)
        cols.append(idxf)
        d = jnp.where(col == idxf, 1e30, d)
    cols.append(jnp.zeros((ROWT, 2), jnp.float32))
    o_ref[...] = jnp.concatenate(cols, axis=1).astype(jnp.int32)


def _knn(pos):
    pos_pad = jnp.zeros((NPAD, 3), jnp.float32).at[:N].set(pos)
    n2c = jnp.sum(pos_pad * pos_pad, axis=1)[None, :]
    out = pl.pallas_call(
        _knn_body,
        grid=(NPAD // ROWT,),
        in_specs=[
            pl.BlockSpec((ROWT, 3), lambda i: (i, 0)),
            pl.BlockSpec((3, NPAD), lambda i: (0, 0)),
            pl.BlockSpec((1, NPAD), lambda i: (0, 0)),
        ],
        out_specs=pl.BlockSpec((ROWT, 8), lambda i: (i, 0)),
        out_shape=jax.ShapeDtypeStruct((NPAD, 8), jnp.int32),
    )(pos_pad, pos_pad.T, n2c)
    return out[:N, : K + 1]


def _dense_bn_body(x_ref, w_ref, b_ref, o_ref):
    o_ref[...] = jax.nn.relu(_bn(_mm(x_ref[...], w_ref[...]) + b_ref[...]))


def _dense_bn(x, w, b):
    return pl.pallas_call(
        _dense_bn_body,
        out_shape=jax.ShapeDtypeStruct((x.shape[0], w.shape[1]), jnp.float32),
    )(x, w, b[None, :])


def _pre_tb_body(x_ref, iw_ref, ib_ref, ws_ref, wd_ref, wv_ref,
                 asrc_ref, adst_ref, xv_ref):
    xr = jax.nn.relu(_mm(x_ref[...], iw_ref[...]) + ib_ref[...])
    asrc_ref[...] = _mm(xr, ws_ref[...])
    adst_ref[...] = _mm(xr, wd_ref[...])
    xv_ref[...] = _mm(xr, wv_ref[...])


def _pre_tb(x, p):
    c = p['in_W'].shape[1]
    sh = jax.ShapeDtypeStruct((x.shape[0], c), jnp.float32)
    return pl.pallas_call(
        _pre_tb_body,
        out_shape=(sh, sh, sh),
    )(x, p['in_W'], p['in_b'][None, :], p['W_src'], p['W_dst'], p['W_val'])


def _edge_a_body(dpos_ref, asrc_ref, adst_ref, xv_ref,
                 pw0_ref, pb0_ref, pw1_ref, pb1_ref,
                 aw0_ref, ab0_ref, aw1_ref, ab1_ref,
                 a_ref, vals_ref):
    h = jax.nn.relu(_mm(dpos_ref[...], pw0_ref[...]) + pb0_ref[...])
    delta = jax.nn.relu(_mm(h, pw1_ref[...]) + pb1_ref[...])
    u = adst_ref[...] - asrc_ref[...] + delta
    h2 = jax.nn.relu(_mm(u, aw0_ref[...]) + ab0_ref[...])
    a_ref[...] = jax.nn.relu(_mm(h2, aw1_ref[...]) + ab1_ref[...])
    vals_ref[...] = xv_ref[...] + delta


def _edge_a(dpos, asrc_g, adst_g, xv_g, p):
    c = p['in_W'].shape[1]
    epad = dpos.shape[0]
    blk = lambda w: pl.BlockSpec((EDGET, w), lambda i: (i, 0))
    full = lambda a: pl.BlockSpec(a.shape, lambda i: (0, 0))
    ws = [p['pW0'], p['pb0'][None, :], p['pW1'], p['pb1'][None, :],
          p['aW0'], p['ab0'][None, :], p['aW1'], p['ab1'][None, :]]
    sh = jax.ShapeDtypeStruct((epad, c), jnp.float32)
    return pl.pallas_call(
        _edge_a_body,
        grid=(epad // EDGET,),
        in_specs=[blk(3), blk(c), blk(c), blk(c)] + [full(w) for w in ws],
        out_specs=(blk(c), blk(c)),
        out_shape=(sh, sh),
    )(dpos, asrc_g, adst_g, xv_g, *ws)


def _edge_ex_body(a_ref, amax_ref, m_ref, vals_ref, ex_ref, exv_ref):
    ex = jnp.exp(a_ref[...] - amax_ref[...]) * m_ref[...]
    ex_ref[...] = ex
    exv_ref[...] = ex * vals_ref[...]


def _edge_ex(a, amax_g, m, vals):
    epad, c = a.shape
    blk = lambda w: pl.BlockSpec((EDGET, w), lambda i: (i, 0))
    sh = jax.ShapeDtypeStruct((epad, c), jnp.float32)
    return pl.pallas_call(
        _edge_ex_body,
        grid=(epad // EDGET,),
        in_specs=[blk(c), blk(c), blk(1), blk(c)],
        out_specs=(blk(c), blk(c)),
        out_shape=(sh, sh),
    )(a, amax_g, m[:, None], vals)


def _mid_body(num_ref, den_ref, ow_ref, ob_ref, tw_ref, tb_ref, o_ref):
    conv = num_ref[...] / (den_ref[...] + 1e-16)
    x1 = jax.nn.relu(_mm(conv, ow_ref[...]) + ob_ref[...])
    o_ref[...] = jax.nn.relu(_bn(_mm(x1, tw_ref[...]) + tb_ref[...]))


def _final_body(num_ref, den_ref, ow_ref, ob_ref,
                w0_ref, b0_ref, w1_ref, b1_ref, w2_ref, b2_ref, o_ref):
    conv = num_ref[...] / (den_ref[...] + 1e-16)
    x3 = jax.nn.relu(_mm(conv, ow_ref[...]) + ob_ref[...])
    g = jnp.mean(x3, axis=0, keepdims=True)
    h = jax.nn.relu(_mm(g, w0_ref[...]) + b0_ref[...])
    h = jax.nn.relu(_mm(h, w1_ref[...]) + b1_ref[...])
    o_ref[...] = _mm(h, w2_ref[...]) + b2_ref[...]


def _pt_conv_block(x, pos, src, dst, p):
    n = x.shape[0]
    loops = jnp.arange(n, dtype=src.dtype)
    m = jnp.concatenate([(src != dst).astype(jnp.float32),
                         jnp.ones((n,), jnp.float32)])
    s = jnp.concatenate([src, loops])
    t = jnp.concatenate([dst, loops])
    e = s.shape[0]
    epad = ((e + EDGET - 1) // EDGET) * EDGET
    s = jnp.concatenate([s, jnp.zeros((epad - e,), s.dtype)])
    t = jnp.concatenate([t, jnp.full((epad - e,), n, t.dtype)])
    m = jnp.concatenate([m, jnp.zeros((epad - e,), jnp.float32)])

    a_src, a_dst, xv = _pre_tb(x, p)
    dpos = pos[jnp.minimum(t, n - 1)] - pos[s]
    a, vals = _edge_a(dpos, a_src[s], a_dst[jnp.minimum(t, n - 1)], xv[s], p)
    amax = jax.ops.segment_max(a, t, num_segments=n + 1)
    amax = jnp.where(jnp.isfinite(amax), amax, 0.0)
    ex, exv = _edge_ex(a, amax[t], m, vals)
    den = jax.ops.segment_sum(ex, t, num_segments=n + 1)[:n]
    num = jax.ops.segment_sum(exv, t, num_segments=n + 1)[:n]
    return num, den


@jax.jit
def _forward_impl(x, pos, edge_index, params):
    n = N
    h0 = _dense_bn(x, params['mlp_in_W'], params['mlp_in_b'])

    p1 = params['tb1']
    num1, den1 = _pt_conv_block(h0, pos, edge_index[0], edge_index[1], p1)
    x_td = pl.pallas_call(
        _mid_body,
        out_shape=jax.ShapeDtypeStruct((n, params['td_W'].shape[1]),
                                       jnp.float32),
    )(num1, den1, p1['out_W'], p1['out_b'][None, :],
      params['td_W'], params['td_b'][None, :])

    idx6 = _knn(pos)
    x2 = jnp.max(x_td[idx6[:, :K]], axis=1)

    p2 = params['tb2']
    src2 = idx6[:, 1:K + 1].reshape(-1)
    dst2 = jnp.repeat(jnp.arange(n, dtype=src2.dtype), K)
    num2, den2 = _pt_conv_block(x2, pos, src2, dst2, p2)

    out = pl.pallas_call(
        _final_body,
        out_shape=jax.ShapeDtypeStruct((1, 10), jnp.float32),
    )(num2, den2, p2['out_W'], p2['out_b'][None, :],
      params['oW0'], params['ob0'][None, :],
      params['oW1'], params['ob1'][None, :],
      params['oW2'], params['ob2'][None, :])
    return out


def kernel(x, pos, edge_index, params):
    return _forward_impl(x, pos, edge_index, params)
